# Initial kernel scaffold; baseline (speedup 1.0000x reference)
import functools

import jax
import jax.numpy as jnp
from jax import lax
from jax.experimental import pallas as pl
from jax.experimental.pallas import tpu as pltpu
from jax.experimental.pallas import tpu_sc as plsc

_NC = 2
_NS = 16
_NW = _NC * _NS
_CH = 80


def _tc_matmul(x, w_ext, bias_row):
    n, d = x.shape
    dw = w_ext.shape[1]
    blk = 400

    def body(x_ref, w_ref, b_ref, o_ref):
        o_ref[...] = (
            jnp.dot(x_ref[...], w_ref[...], preferred_element_type=jnp.float32)
            + b_ref[...]
        )

    return pl.pallas_call(
        body,
        grid=(n // blk,),
        in_specs=[
            pl.BlockSpec((blk, d), lambda i: (i, 0)),
            pl.BlockSpec((d, dw), lambda i: (0, 0)),
            pl.BlockSpec((1, dw), lambda i: (0, 0)),
        ],
        out_specs=pl.BlockSpec((blk, dw), lambda i: (i, 0)),
        out_shape=jax.ShapeDtypeStruct((n, dw), jnp.float32),
    )(x, w_ext, bias_row)


def _sc_edge_weights(src2, dst2, a, b):
    r_tot, ch = src2.shape
    n = a.shape[0]
    rpw = r_tot // _NW
    seg = 2000
    mesh = plsc.VectorSubcoreMesh(core_axis_name="c", subcore_axis_name="s")

    @functools.partial(
        pl.kernel,
        out_type=(
            jax.ShapeDtypeStruct((r_tot, ch), jnp.float32),
            jax.ShapeDtypeStruct((_NC, n), jnp.float32),
        ),
        mesh=mesh,
        scratch_types=(
            pltpu.VMEM((n,), jnp.float32),
            pltpu.VMEM((n,), jnp.float32),
            pltpu.VMEM((rpw, ch), jnp.int32),
            pltpu.VMEM((rpw, ch), jnp.int32),
            pltpu.VMEM((rpw, ch), jnp.float32),
            pltpu.VMEM((2000,), jnp.float32),
            pltpu.VMEM_SHARED((n,), jnp.float32),
            pltpu.SemaphoreType.DMA,
        ),
    )
    def kern(src_hbm, dst_hbm, a_hbm, b_hbm, ew_hbm, degp_hbm,
             a_v, b_v, src_v, dst_v, ew_v, stage_v, deg_sh, sem):
        cid = lax.axis_index("c")
        sid = lax.axis_index("s")
        wid = cid * _NS + sid
        row0 = wid * rpw

        @pl.when(sid < n // seg)
        def _():
            for t in range(seg // 16):
                stage_v[pl.ds(t * 16, 16)] = jnp.zeros((16,), jnp.float32)
            pltpu.sync_copy(stage_v, deg_sh.at[pl.ds(sid * seg, seg)])

        pltpu.sync_copy(a_hbm, a_v)
        pltpu.sync_copy(b_hbm, b_v)
        pltpu.sync_copy(src_hbm.at[pl.ds(row0, rpw)], src_v)
        pltpu.sync_copy(dst_hbm.at[pl.ds(row0, rpw)], dst_v)
        plsc.subcore_barrier()

        def chunk(r_):
            for g in range(ch // 16):
                sv = src_v[r_, pl.ds(g * 16, 16)]
                dv = dst_v[r_, pl.ds(g * 16, 16)]
                av = plsc.load_gather(a_v, [sv])
                bv = plsc.load_gather(b_v, [dv])
                ew = 1.0 / (1.0 + jnp.exp(-(av + bv)))
                ew_v[r_, pl.ds(g * 16, 16)] = ew

        pl.loop(0, rpw)(chunk)

        def fire(r0):
            descs = [
                pltpu.async_copy(
                    ew_v.at[r0 + j], deg_sh.at[dst_v.at[r0 + j]], sem, add=True
                )
                for j in range(25)
            ]
            for de in descs:
                de.wait()

        pl.loop(0, rpw, step=25)(fire)

        pltpu.sync_copy(ew_v, ew_hbm.at[pl.ds(row0, rpw)])
        plsc.subcore_barrier()

        @pl.when(sid < n // seg)
        def _():
            pltpu.sync_copy(deg_sh.at[pl.ds(sid * seg, seg)], stage_v)
            pltpu.sync_copy(stage_v, degp_hbm.at[cid, pl.ds(sid * seg, seg)])

    return kern(src2, dst2, a, b)


def _sc_scatter(src2, dst2, ew2, y):
    r_tot, ch = src2.shape
    n, d = y.shape
    rpw = r_tot // _NW
    rows_per_tile = n // _NS
    stage_rows = 125
    mesh = plsc.VectorSubcoreMesh(core_axis_name="c", subcore_axis_name="s")

    @functools.partial(
        pl.kernel,
        out_type=jax.ShapeDtypeStruct((_NC, n, d), jnp.float32),
        mesh=mesh,
        scratch_types=(
            pltpu.VMEM((rpw, ch), jnp.int32),
            pltpu.VMEM((rpw, ch), jnp.int32),
            pltpu.VMEM((rpw, ch), jnp.float32),
            pltpu.VMEM((2, ch, d), jnp.float32),
            pltpu.VMEM((125, d), jnp.float32),
            pltpu.VMEM_SHARED((n, d), jnp.float32),
            pltpu.SemaphoreType.DMA,
        ),
    )
    def kern(src_hbm, dst_hbm, ew_hbm, y_hbm, msg_hbm,
             src_v, dst_v, ew_v, rows_v, stage_v, acc_sh, gsem):
        cid = lax.axis_index("c")
        sid = lax.axis_index("s")
        wid = cid * _NS + sid
        row0 = wid * rpw

        def zrow(t):
            for j in range(d // 16):
                stage_v[t, pl.ds(j * 16, 16)] = jnp.zeros((16,), jnp.float32)

        pl.loop(0, 125)(zrow)
        for q in range(rows_per_tile // 125):
            pltpu.sync_copy(
                stage_v,
                acc_sh.at[pl.ds(sid * rows_per_tile + q * 125, 125)],
            )

        pltpu.sync_copy(src_hbm.at[pl.ds(row0, rpw)], src_v)
        pltpu.sync_copy(dst_hbm.at[pl.ds(row0, rpw)], dst_v)
        pltpu.sync_copy(ew_hbm.at[pl.ds(row0, rpw)], ew_v)
        plsc.subcore_barrier()

        def process(rr, buf):
            pltpu.make_async_copy(
                y_hbm.at[src_v.at[rr]], rows_v.at[buf], gsem
            ).wait()

            @pl.when(rr + 1 < rpw)
            def _():
                pltpu.async_copy(
                    y_hbm.at[src_v.at[rr + 1]], rows_v.at[1 - buf], gsem
                )

            def sedge(e):
                s = ew_v[rr, e]
                sv = jnp.full((16,), s, jnp.float32)
                for j in range(d // 16):
                    rows_v[buf, e, pl.ds(j * 16, 16)] = (
                        rows_v[buf, e, pl.ds(j * 16, 16)] * sv
                    )

            pl.loop(0, ch, unroll=4)(sedge)

            pltpu.sync_copy(rows_v.at[buf], acc_sh.at[dst_v.at[rr]], add=True)

        pltpu.async_copy(y_hbm.at[src_v.at[0]], rows_v.at[0], gsem)

        def step2(r_):
            process(r_, 0)
            process(r_ + 1, 1)

        pl.loop(0, rpw - 1, step=2)(step2)
        process(rpw - 1, 0)

        plsc.subcore_barrier()
        for q in range(rows_per_tile // 125):
            r0 = sid * rows_per_tile + q * 125
            pltpu.sync_copy(acc_sh.at[pl.ds(r0, 125)], stage_v)
            pltpu.sync_copy(stage_v, msg_hbm.at[cid, pl.ds(r0, 125)])

    return kern(src2, dst2, ew2, y)


def _tc_scale(degp_t, x_lin):
    n, d = x_lin.shape
    blk = 400

    def body(p_ref, xl_ref, y_ref):
        deg = 1.0 + p_ref[:, 0:1] + p_ref[:, 1:2]
        y_ref[...] = lax.rsqrt(deg) * xl_ref[...]

    return pl.pallas_call(
        body,
        grid=(n // blk,),
        in_specs=[
            pl.BlockSpec((blk, 2), lambda i: (i, 0)),
            pl.BlockSpec((blk, d), lambda i: (i, 0)),
        ],
        out_specs=pl.BlockSpec((blk, d), lambda i: (i, 0)),
        out_shape=jax.ShapeDtypeStruct((n, d), jnp.float32),
    )(degp_t, x_lin)


def _tc_combine(degp_t, msg, y, bias):
    n, d = y.shape
    blk = 400

    def body(p_ref, m_ref, y_ref, b_ref, o_ref):
        deg = 1.0 + p_ref[:, 0:1] + p_ref[:, 1:2]
        dis = lax.rsqrt(deg)
        o_ref[...] = dis * (m_ref[0] + m_ref[1] + y_ref[...]) + b_ref[...]

    return pl.pallas_call(
        body,
        grid=(n // blk,),
        in_specs=[
            pl.BlockSpec((blk, 2), lambda i: (i, 0)),
            pl.BlockSpec((2, blk, d), lambda i: (0, i, 0)),
            pl.BlockSpec((blk, d), lambda i: (i, 0)),
            pl.BlockSpec((1, d), lambda i: (0, 0)),
        ],
        out_specs=pl.BlockSpec((blk, d), lambda i: (i, 0)),
        out_shape=jax.ShapeDtypeStruct((n, d), jnp.float32),
    )(degp_t, msg, y, bias)


def kernel(x, edge_index, W_pred, b_pred, W_gcn, b_gcn):
    n, d = x.shape
    e = edge_index.shape[1]

    src2 = edge_index[0].astype(jnp.int32).reshape(e // _CH, _CH)
    dst2 = edge_index[1].astype(jnp.int32).reshape(e // _CH, _CH)

    w_ext = jnp.concatenate(
        [W_gcn, W_pred[:d], W_pred[d:], jnp.zeros((d, d - 2), jnp.float32)],
        axis=1,
    )
    bias_row = jnp.zeros((1, 2 * d), jnp.float32).at[0, d + 1].set(b_pred[0])

    xe = _tc_matmul(x, w_ext, bias_row)
    x_lin = xe[:, :d]
    a = xe[:, d]
    b = xe[:, d + 1]

    ew2, degp = _sc_edge_weights(src2, dst2, a, b)
    degp_t = degp.T

    y = _tc_scale(degp_t, x_lin)
    msg = _sc_scatter(src2, dst2, ew2, y)
    out = _tc_combine(degp_t, msg, y, b_gcn.reshape(1, d))
    return out


# speedup vs baseline: 24.5146x; 24.5146x over previous
import functools

import jax
import jax.numpy as jnp
from jax import lax
from jax.experimental import pallas as pl
from jax.experimental.pallas import tpu as pltpu
from jax.experimental.pallas import tpu_sc as plsc

_NC = 2
_NS = 16
_NW = _NC * _NS
_CH = 80


def _tc_matmul(x, w_ext, bias_row):
    n, d = x.shape
    dw = w_ext.shape[1]
    blk = 400

    def body(x_ref, w_ref, b_ref, o_ref):
        o_ref[...] = (
            jnp.dot(x_ref[...], w_ref[...], preferred_element_type=jnp.float32)
            + b_ref[...]
        )

    return pl.pallas_call(
        body,
        grid=(n // blk,),
        in_specs=[
            pl.BlockSpec((blk, d), lambda i: (i, 0)),
            pl.BlockSpec((d, dw), lambda i: (0, 0)),
            pl.BlockSpec((1, dw), lambda i: (0, 0)),
        ],
        out_specs=pl.BlockSpec((blk, dw), lambda i: (i, 0)),
        out_shape=jax.ShapeDtypeStruct((n, dw), jnp.float32),
    )(x, w_ext, bias_row)


def _row_partition(n_rows, n_tiles, max_chunk):
    per = -(-n_rows // n_tiles)
    per = ((per + 7) // 8) * 8
    parts = []
    base = 0
    for _ in range(n_tiles):
        cnt = max(0, min(per, n_rows - base))
        sizes = []
        left = cnt
        while left > 0:
            sz = min(max_chunk, left)
            sizes.append(sz)
            left -= sz
        parts.append((base, sizes))
        base += cnt
    return parts


def _sc_edge_weights(src3, dst3, a, b):
    _, rpw, ch = src3.shape
    n = a.shape[0]
    seg = 2000
    mesh = plsc.VectorSubcoreMesh(core_axis_name="c", subcore_axis_name="s")

    @functools.partial(
        pl.kernel,
        out_type=(
            jax.ShapeDtypeStruct((_NW, rpw, ch), jnp.float32),
            jax.ShapeDtypeStruct((_NC * n,), jnp.float32),
        ),
        mesh=mesh,
        scratch_types=(
            pltpu.VMEM((n,), jnp.float32),
            pltpu.VMEM((n,), jnp.float32),
            pltpu.VMEM((rpw, ch), jnp.int32),
            pltpu.VMEM((rpw, ch), jnp.int32),
            pltpu.VMEM((rpw, ch), jnp.float32),
            pltpu.VMEM((seg,), jnp.float32),
            pltpu.VMEM_SHARED((n,), jnp.float32),
            pltpu.SemaphoreType.DMA,
        ),
        compiler_params=pltpu.CompilerParams(needs_layout_passes=False),
    )
    def kern(src_hbm, dst_hbm, a_hbm, b_hbm, ew_hbm, degp_hbm,
             a_v, b_v, src_v, dst_v, ew_v, stage_v, deg_sh, sem):
        cid = lax.axis_index("c")
        sid = lax.axis_index("s")
        wid = cid * _NS + sid

        @pl.when(sid < n // seg)
        def _():
            for t in range(seg // 16):
                stage_v[pl.ds(t * 16, 16)] = jnp.zeros((16,), jnp.float32)
            pltpu.sync_copy(stage_v, deg_sh.at[pl.ds(sid * seg, seg)])

        pltpu.sync_copy(a_hbm, a_v)
        pltpu.sync_copy(b_hbm, b_v)
        pltpu.sync_copy(src_hbm.at[wid], src_v)
        pltpu.sync_copy(dst_hbm.at[wid], dst_v)
        plsc.subcore_barrier()

        def chunk(r_):
            for g in range(ch // 16):
                sv = src_v[r_, pl.ds(g * 16, 16)]
                dv = dst_v[r_, pl.ds(g * 16, 16)]
                av = plsc.load_gather(a_v, [sv])
                bv = plsc.load_gather(b_v, [dv])
                ew = 1.0 / (1.0 + jnp.exp(-(av + bv)))
                ew_v[r_, pl.ds(g * 16, 16)] = ew

        pl.loop(0, rpw)(chunk)

        def fire(r0):
            descs = [
                pltpu.async_copy(
                    ew_v.at[r0 + j], deg_sh.at[dst_v.at[r0 + j]], sem, add=True
                )
                for j in range(25)
            ]
            for de in descs:
                de.wait()

        pl.loop(0, rpw, step=25)(fire)

        pltpu.sync_copy(ew_v, ew_hbm.at[wid])
        plsc.subcore_barrier()

        @pl.when(sid < n // seg)
        def _():
            pltpu.sync_copy(deg_sh.at[pl.ds(sid * seg, seg)], stage_v)
            pltpu.sync_copy(
                stage_v, degp_hbm.at[pl.ds(cid * n + sid * seg, seg)]
            )

    return kern(src3, dst3, a, b)


def _sc_scatter(src3, dst3, ew3, y2):
    _, rpw, ch = src3.shape
    n2, dh = y2.shape
    n = n2 // 2
    stage_rows = 128
    parts = _row_partition(n, _NS, stage_rows)
    mesh = plsc.VectorSubcoreMesh(core_axis_name="c", subcore_axis_name="s")

    @functools.partial(
        pl.kernel,
        out_type=jax.ShapeDtypeStruct((_NC, n, dh), jnp.float32),
        mesh=mesh,
        scratch_types=(
            pltpu.VMEM((rpw, ch), jnp.int32),
            pltpu.VMEM((rpw, ch), jnp.int32),
            pltpu.VMEM((rpw, ch), jnp.float32),
            pltpu.VMEM((2, ch, dh), jnp.float32),
            pltpu.VMEM((stage_rows, dh), jnp.float32),
            pltpu.VMEM_SHARED((n, dh), jnp.float32),
            pltpu.SemaphoreType.DMA,
        ),
        compiler_params=pltpu.CompilerParams(
            needs_layout_passes=False, use_tc_tiling_on_sc=False
        ),
    )
    def kern(src_hbm, dst_hbm, ew_hbm, y_hbm, msg_hbm,
             src_v, dst_v, ew_v, rows_v, stage_v, acc_sh, gsem):
        cid = lax.axis_index("c")
        sid = lax.axis_index("s")

        def zrow(t):
            for j in range(dh // 16):
                stage_v[t, pl.ds(j * 16, 16)] = jnp.zeros((16,), jnp.float32)

        pl.loop(0, stage_rows)(zrow)
        for t, (base, sizes) in enumerate(parts):
            @pl.when(sid == t)
            def _(base=base, sizes=sizes):
                off = 0
                for sz in sizes:
                    pltpu.sync_copy(
                        stage_v.at[pl.ds(0, sz)],
                        acc_sh.at[pl.ds(base + off, sz)],
                    )
                    off += sz

        pltpu.sync_copy(src_hbm.at[sid], src_v)
        pltpu.sync_copy(dst_hbm.at[sid], dst_v)
        pltpu.sync_copy(ew_hbm.at[sid], ew_v)

        def remap(r_):
            for g in range(ch // 16):
                sv = src_v[r_, pl.ds(g * 16, 16)]
                src_v[r_, pl.ds(g * 16, 16)] = sv * 2 + cid

        pl.loop(0, rpw)(remap)
        plsc.subcore_barrier()

        def process(rr, buf):
            pltpu.make_async_copy(
                y_hbm.at[src_v.at[rr]], rows_v.at[buf], gsem
            ).wait()

            @pl.when(rr + 1 < rpw)
            def _():
                pltpu.async_copy(
                    y_hbm.at[src_v.at[rr + 1]], rows_v.at[1 - buf], gsem
                )

            def sgroup(g):
                ewg = ew_v[rr, pl.ds(g * 16, 16)]
                for l in range(16):
                    sv = jnp.full((16,), ewg[l], jnp.float32)
                    e = g * 16 + l
                    for j in range(dh // 16):
                        rows_v[buf, e, pl.ds(j * 16, 16)] = (
                            rows_v[buf, e, pl.ds(j * 16, 16)] * sv
                        )

            pl.loop(0, ch // 16)(sgroup)

            pltpu.sync_copy(rows_v.at[buf], acc_sh.at[dst_v.at[rr]], add=True)

        pltpu.async_copy(y_hbm.at[src_v.at[0]], rows_v.at[0], gsem)

        def step2(r_):
            process(r_, 0)
            process(r_ + 1, 1)

        pl.loop(0, rpw, step=2)(step2)

        plsc.subcore_barrier()
        for t, (base, sizes) in enumerate(parts):
            @pl.when(sid == t)
            def _(base=base, sizes=sizes):
                off = 0
                for sz in sizes:
                    pltpu.sync_copy(
                        acc_sh.at[pl.ds(base + off, sz)],
                        stage_v.at[pl.ds(0, sz)],
                    )
                    pltpu.sync_copy(
                        stage_v.at[pl.ds(0, sz)],
                        msg_hbm.at[cid, pl.ds(base + off, sz)],
                    )
                    off += sz

    return kern(src3, dst3, ew3, y2)


def _tc_scale(degp_t, x_lin):
    n, d = x_lin.shape
    blk = 400

    def body(p_ref, xl_ref, y_ref):
        deg = 1.0 + p_ref[:, 0:1] + p_ref[:, 1:2]
        y_ref[...] = lax.rsqrt(deg) * xl_ref[...]

    return pl.pallas_call(
        body,
        grid=(n // blk,),
        in_specs=[
            pl.BlockSpec((blk, 2), lambda i: (i, 0)),
            pl.BlockSpec((blk, d), lambda i: (i, 0)),
        ],
        out_specs=pl.BlockSpec((blk, d), lambda i: (i, 0)),
        out_shape=jax.ShapeDtypeStruct((n, d), jnp.float32),
    )(degp_t, x_lin)


def _tc_combine(degp_t, msg, y, bias):
    n, d = y.shape
    blk = 400

    def body(p_ref, m_ref, y_ref, b_ref, o_ref):
        deg = 1.0 + p_ref[:, 0:1] + p_ref[:, 1:2]
        dis = lax.rsqrt(deg)
        m_full = jnp.concatenate([m_ref[0], m_ref[1]], axis=1)
        o_ref[...] = dis * (m_full + y_ref[...]) + b_ref[...]

    return pl.pallas_call(
        body,
        grid=(n // blk,),
        in_specs=[
            pl.BlockSpec((blk, 2), lambda i: (i, 0)),
            pl.BlockSpec((2, blk, d // 2), lambda i: (0, i, 0)),
            pl.BlockSpec((blk, d), lambda i: (i, 0)),
            pl.BlockSpec((1, d), lambda i: (0, 0)),
        ],
        out_specs=pl.BlockSpec((blk, d), lambda i: (i, 0)),
        out_shape=jax.ShapeDtypeStruct((n, d), jnp.float32),
    )(degp_t, msg, y, bias)


def kernel(x, edge_index, W_pred, b_pred, W_gcn, b_gcn):
    n, d = x.shape
    e = edge_index.shape[1]
    rpw = e // (_NW * _CH)
    rpw2 = e // (_NS * _CH)

    src = edge_index[0].astype(jnp.int32)
    dst = edge_index[1].astype(jnp.int32)
    src3 = src.reshape(_NW, rpw, _CH)
    dst3 = dst.reshape(_NW, rpw, _CH)

    w_ext = jnp.concatenate(
        [W_gcn, W_pred[:d], W_pred[d:], jnp.zeros((d, d - 2), jnp.float32)],
        axis=1,
    )
    bias_row = jnp.zeros((1, 2 * d), jnp.float32).at[0, d + 1].set(b_pred[0])

    xe = _tc_matmul(x, w_ext, bias_row)
    x_lin = xe[:, :d]
    a = xe[:, d]
    b = xe[:, d + 1]

    ew3, degp = _sc_edge_weights(src3, dst3, a, b)
    degp_t = degp.reshape(_NC, n).T

    y = _tc_scale(degp_t, x_lin)
    msg = _sc_scatter(
        src.reshape(_NS, rpw2, _CH),
        dst.reshape(_NS, rpw2, _CH),
        ew3.reshape(_NS, rpw2, _CH),
        y.reshape(2 * n, d // 2),
    )
    out = _tc_combine(degp_t, msg, y, b_gcn.reshape(1, d))
    return out
